# baseline (speedup 1.0000x reference)
import functools

import jax
import jax.numpy as jnp
from jax import lax
from jax.experimental import pallas as pl
from jax.experimental.pallas import tpu as pltpu
from jax.experimental.pallas import tpu_sc as plsc

_BATCH = 16384
_HIST = 50
_D = 64
_B = _BATCH * _HIST
_NC = 2
_NS = 16
_NW = _NC * _NS
_B_PER_W = _B // _NW
_IDX_MINOR = 128
_STREAMS_PER_GROUP = 2
_NBUF = 4
_GROUP = _IDX_MINOR * _STREAMS_PER_GROUP
_N_GROUPS = _B_PER_W // _GROUP
_ROWS_PER_W = _B_PER_W // _IDX_MINOR


def _make_emb_kernel():
  mesh = plsc.VectorSubcoreMesh(core_axis_name="c", subcore_axis_name="s")

  @functools.partial(
      pl.kernel,
      mesh=mesh,
      compiler_params=pltpu.CompilerParams(use_tc_tiling_on_sc=False),
      out_type=jax.ShapeDtypeStruct((_B, _D), jnp.float32),
      scratch_types=(
          [pltpu.VMEM((_ROWS_PER_W, _IDX_MINOR), jnp.int32)]
          + [pltpu.VMEM((_GROUP, _D), jnp.float32) for _ in range(_NBUF)]
          + [pltpu.SemaphoreType.DMA for _ in range(_NBUF)]
      ),
  )
  def emb(idx_hbm, table_hbm, out_hbm, idx_v, *bufs):
    rows = bufs[:_NBUF]
    gsem = bufs[_NBUF:]
    wid = lax.axis_index("s") * _NC + lax.axis_index("c")
    base = wid * _B_PER_W

    pltpu.sync_copy(idx_hbm.at[pl.ds(wid * _ROWS_PER_W, _ROWS_PER_W)], idx_v)

    def fire(g, b):
      for s in range(_STREAMS_PER_GROUP):
        row = g * _STREAMS_PER_GROUP + s
        pltpu.async_copy(
            table_hbm.at[idx_v.at[row]],
            rows[b].at[pl.ds(s * _IDX_MINOR, _IDX_MINOR)],
            gsem[b],
        )

    def drain(b):
      for s in range(_STREAMS_PER_GROUP):
        pltpu.make_async_copy(
            out_hbm.at[pl.ds(0, _IDX_MINOR)],
            rows[b].at[pl.ds(s * _IDX_MINOR, _IDX_MINOR)],
            gsem[b],
        ).wait()

    def store(g, b):
      pltpu.sync_copy(rows[b], out_hbm.at[pl.ds(base + g * _GROUP, _GROUP)])

    for b in range(_NBUF):
      fire(b, b)

    def body(h, carry):
      for b in range(_NBUF):
        g = h * _NBUF + b
        drain(b)
        store(g, b)
        fire(g + _NBUF, b)
      return carry

    lax.fori_loop(0, _N_GROUPS // _NBUF - 1, body, 0)

    for b in range(_NBUF):
      g = _N_GROUPS - _NBUF + b
      drain(b)
      store(g, b)

  return emb


_EMB = _make_emb_kernel()


@jax.jit
def kernel(x, lut_weight):
  idx = x.reshape(_B // _IDX_MINOR, _IDX_MINOR).astype(jnp.int32)
  out = _EMB(idx, lut_weight)
  return out.reshape(_BATCH, _HIST, _D)


# speedup vs baseline: 1.0040x; 1.0040x over previous
import functools

import jax
import jax.numpy as jnp
from jax import lax
from jax.experimental import pallas as pl
from jax.experimental.pallas import tpu as pltpu
from jax.experimental.pallas import tpu_sc as plsc

_BATCH = 16384
_HIST = 50
_D = 64
_B = _BATCH * _HIST
_NC = 2
_NS = 16
_NW = _NC * _NS
_B_PER_W = _B // _NW
_GROUP = 512
_NBUF = 3
_N_GROUPS = _B_PER_W // _GROUP


def _make_emb_kernel():
  mesh = plsc.VectorSubcoreMesh(core_axis_name="c", subcore_axis_name="s")

  @functools.partial(
      pl.kernel,
      mesh=mesh,
      compiler_params=pltpu.CompilerParams(use_tc_tiling_on_sc=False),
      out_type=jax.ShapeDtypeStruct((_B, _D), jnp.float32),
      scratch_types=(
          [pltpu.VMEM((_B_PER_W,), jnp.int32)]
          + [pltpu.VMEM((_GROUP, _D), jnp.float32) for _ in range(_NBUF)]
          + [pltpu.SemaphoreType.DMA for _ in range(_NBUF)]
      ),
  )
  def emb(idx_hbm, table_hbm, out_hbm, idx_v, *bufs):
    rows = bufs[:_NBUF]
    gsem = bufs[_NBUF:]
    wid = lax.axis_index("s") * _NC + lax.axis_index("c")
    base = wid * _B_PER_W

    pltpu.sync_copy(idx_hbm.at[pl.ds(base, _B_PER_W)], idx_v)

    def fire(g, b):
      pltpu.async_copy(
          table_hbm.at[idx_v.at[pl.ds(g * _GROUP, _GROUP)]],
          rows[b],
          gsem[b],
      )

    def drain(b):
      pltpu.make_async_copy(
          out_hbm.at[pl.ds(0, _GROUP)],
          rows[b],
          gsem[b],
      ).wait()

    def store(g, b):
      pltpu.sync_copy(rows[b], out_hbm.at[pl.ds(base + g * _GROUP, _GROUP)])

    for b in range(_NBUF):
      fire(b, b)

    n_main = (_N_GROUPS - _NBUF) // _NBUF * _NBUF

    def body(h, carry):
      for b in range(_NBUF):
        g = h * _NBUF + b
        drain(b)
        store(g, b)
        fire(g + _NBUF, b)
      return carry

    lax.fori_loop(0, n_main // _NBUF, body, 0)

    for g in range(n_main, _N_GROUPS - _NBUF):
      b = g % _NBUF
      drain(b)
      store(g, b)
      fire(g + _NBUF, b)

    for g in range(_N_GROUPS - _NBUF, _N_GROUPS):
      b = g % _NBUF
      drain(b)
      store(g, b)

  return emb


_EMB = _make_emb_kernel()


@jax.jit
def kernel(x, lut_weight):
  idx = x.reshape(_B).astype(jnp.int32)
  out = _EMB(idx, lut_weight)
  return out.reshape(_BATCH, _HIST, _D)
